# Initial kernel scaffold; baseline (speedup 1.0000x reference)
import functools

import jax
import jax.numpy as jnp
from jax import lax
from jax.experimental import pallas as pl
from jax.experimental.pallas import tpu as pltpu
from jax.experimental.pallas import tpu_sc as plsc

NC, NS = 2, 16
NW = NC * NS
SUB = 128
SUBS_PER_CHUNK = 4
CHUNK = SUB * SUBS_PER_CHUNK


@functools.partial(jax.jit, static_argnames=("n_chunks",))
def _sc_embed(w2, f2, p2, r2, W_word, W_field, W_pos, W_rpos, n_chunks):
    N = w2.shape[0] * SUB
    WD = W_word.shape[1]
    FD = W_field.shape[1]
    PD = W_pos.shape[1]
    RD = W_rpos.shape[1]
    CD = FD + PD + RD

    mesh = plsc.VectorSubcoreMesh(
        core_axis_name="c", subcore_axis_name="s", num_cores=NC, num_subcores=NS
    )

    @functools.partial(
        pl.kernel,
        mesh=mesh,
        out_type=[
            jax.ShapeDtypeStruct((N, WD), jnp.float32),
            jax.ShapeDtypeStruct((N, CD), jnp.float32),
        ],
        scratch_types=[
            pltpu.VMEM((SUBS_PER_CHUNK, SUB), jnp.int32),
            pltpu.VMEM((SUBS_PER_CHUNK, SUB), jnp.int32),
            pltpu.VMEM((SUBS_PER_CHUNK, SUB), jnp.int32),
            pltpu.VMEM((SUBS_PER_CHUNK, SUB), jnp.int32),
            pltpu.VMEM((CHUNK, WD), jnp.float32),
            pltpu.VMEM((CHUNK, FD), jnp.float32),
            pltpu.VMEM((CHUNK, PD), jnp.float32),
            pltpu.VMEM((CHUNK, RD), jnp.float32),
            pltpu.SemaphoreType.DMA,
        ],
    )
    def k(words_h, fields_h, pos_h, rpos_h, Ww, Wf, Wp, Wr,
          word_out, fp_out, widx, fidx, pidx, ridx,
          wrows, frows, prows, rrows, sem):
        wid = lax.axis_index("s") * NC + lax.axis_index("c")
        row0 = wid * (n_chunks * SUBS_PER_CHUNK)

        def chunk(g, carry):
            r = row0 + g * SUBS_PER_CHUNK
            base = r * SUB
            pltpu.sync_copy(words_h.at[pl.ds(r, SUBS_PER_CHUNK)], widx)
            pltpu.sync_copy(fields_h.at[pl.ds(r, SUBS_PER_CHUNK)], fidx)
            pltpu.sync_copy(pos_h.at[pl.ds(r, SUBS_PER_CHUNK)], pidx)
            pltpu.sync_copy(rpos_h.at[pl.ds(r, SUBS_PER_CHUNK)], ridx)
            cps = []
            for j in range(SUBS_PER_CHUNK):
                o = j * SUB
                cps.append(pltpu.async_copy(
                    Ww.at[widx.at[j]], wrows.at[pl.ds(o, SUB)], sem))
                cps.append(pltpu.async_copy(
                    Wf.at[fidx.at[j]], frows.at[pl.ds(o, SUB)], sem))
                cps.append(pltpu.async_copy(
                    Wp.at[pidx.at[j]], prows.at[pl.ds(o, SUB)], sem))
                cps.append(pltpu.async_copy(
                    Wr.at[ridx.at[j]], rrows.at[pl.ds(o, SUB)], sem))
            for cp in cps:
                cp.wait()
            pltpu.sync_copy(wrows, word_out.at[pl.ds(base, CHUNK)])
            pltpu.sync_copy(frows, fp_out.at[pl.ds(base, CHUNK), pl.ds(0, FD)])
            pltpu.sync_copy(prows, fp_out.at[pl.ds(base, CHUNK), pl.ds(FD, PD)])
            pltpu.sync_copy(rrows, fp_out.at[pl.ds(base, CHUNK), pl.ds(FD + PD, RD)])
            return carry

        lax.fori_loop(0, n_chunks, chunk, 0)

    return k(w2, f2, p2, r2, W_word, W_field, W_pos, W_rpos)


def kernel(words, fields, pos, rpos, W_word, W_field, W_pos, W_rpos):
    B, L = words.shape
    N = B * L
    assert N % (NW * CHUNK) == 0
    n_chunks = N // (NW * CHUNK)

    def prep(ix):
        return ix.reshape(N // SUB, SUB).astype(jnp.int32)

    word_flat, fp_flat = _sc_embed(
        prep(words), prep(fields), prep(pos), prep(rpos),
        W_word, W_field, W_pos, W_rpos, n_chunks)

    WD = W_word.shape[1]
    CD = fp_flat.shape[1]
    return word_flat.reshape(B, L, WD), fp_flat.reshape(B, L, CD)


# speedup vs baseline: 5.5161x; 5.5161x over previous
import functools

import jax
import jax.numpy as jnp
from jax import lax
from jax.experimental import pallas as pl
from jax.experimental.pallas import tpu as pltpu
from jax.experimental.pallas import tpu_sc as plsc

NC, NS = 2, 16
NW = NC * NS
SUB = 128
SUBS_PER_CHUNK = 4
CHUNK = SUB * SUBS_PER_CHUNK


@functools.partial(jax.jit, static_argnames=("n_chunks",))
def _sc_embed(w2, f2, p2, r2, W_word, W_field, W_pos, W_rpos, n_chunks):
    N = w2.shape[0] * SUB
    WD = W_word.shape[1]
    FD = W_field.shape[1]
    PD = W_pos.shape[1]
    RD = W_rpos.shape[1]
    CD = FD + PD + RD

    mesh = plsc.VectorSubcoreMesh(
        core_axis_name="c", subcore_axis_name="s", num_cores=NC, num_subcores=NS
    )

    @functools.partial(
        pl.kernel,
        mesh=mesh,
        out_type=[
            jax.ShapeDtypeStruct((N, WD), jnp.float32),
            jax.ShapeDtypeStruct((N, CD), jnp.float32),
        ],
        scratch_types=[
            pltpu.VMEM((SUBS_PER_CHUNK, SUB), jnp.int32),
            pltpu.VMEM((SUBS_PER_CHUNK, SUB), jnp.int32),
            pltpu.VMEM((SUBS_PER_CHUNK, SUB), jnp.int32),
            pltpu.VMEM((SUBS_PER_CHUNK, SUB), jnp.int32),
            pltpu.VMEM((CHUNK, WD), jnp.float32),
            pltpu.VMEM((CHUNK, FD), jnp.float32),
            pltpu.VMEM((CHUNK, PD), jnp.float32),
            pltpu.VMEM((CHUNK, RD), jnp.float32),
            pltpu.SemaphoreType.DMA,
        ],
        compiler_params=pltpu.CompilerParams(use_tc_tiling_on_sc=False),
    )
    def k(words_h, fields_h, pos_h, rpos_h, Ww, Wf, Wp, Wr,
          word_out, fp_out, widx, fidx, pidx, ridx,
          wrows, frows, prows, rrows, sem):
        wid = lax.axis_index("s") * NC + lax.axis_index("c")
        row0 = wid * (n_chunks * SUBS_PER_CHUNK)

        def chunk(g, carry):
            r = row0 + g * SUBS_PER_CHUNK
            base = r * SUB
            pltpu.sync_copy(words_h.at[pl.ds(r, SUBS_PER_CHUNK)], widx)
            pltpu.sync_copy(fields_h.at[pl.ds(r, SUBS_PER_CHUNK)], fidx)
            pltpu.sync_copy(pos_h.at[pl.ds(r, SUBS_PER_CHUNK)], pidx)
            pltpu.sync_copy(rpos_h.at[pl.ds(r, SUBS_PER_CHUNK)], ridx)
            cps = []
            for j in range(SUBS_PER_CHUNK):
                o = j * SUB
                cps.append(pltpu.async_copy(
                    Ww.at[widx.at[j]], wrows.at[pl.ds(o, SUB)], sem))
                cps.append(pltpu.async_copy(
                    Wf.at[fidx.at[j]], frows.at[pl.ds(o, SUB)], sem))
                cps.append(pltpu.async_copy(
                    Wp.at[pidx.at[j]], prows.at[pl.ds(o, SUB)], sem))
                cps.append(pltpu.async_copy(
                    Wr.at[ridx.at[j]], rrows.at[pl.ds(o, SUB)], sem))
            for cp in cps:
                cp.wait()
            pltpu.sync_copy(wrows, word_out.at[pl.ds(base, CHUNK)])
            pltpu.sync_copy(frows, fp_out.at[pl.ds(base, CHUNK), pl.ds(0, FD)])
            pltpu.sync_copy(prows, fp_out.at[pl.ds(base, CHUNK), pl.ds(FD, PD)])
            pltpu.sync_copy(rrows, fp_out.at[pl.ds(base, CHUNK), pl.ds(FD + PD, RD)])
            return carry

        lax.fori_loop(0, n_chunks, chunk, 0)

    return k(w2, f2, p2, r2, W_word, W_field, W_pos, W_rpos)


def kernel(words, fields, pos, rpos, W_word, W_field, W_pos, W_rpos):
    B, L = words.shape
    N = B * L
    assert N % (NW * CHUNK) == 0
    n_chunks = N // (NW * CHUNK)

    def prep(ix):
        return ix.reshape(N // SUB, SUB).astype(jnp.int32)

    word_flat, fp_flat = _sc_embed(
        prep(words), prep(fields), prep(pos), prep(rpos),
        W_word, W_field, W_pos, W_rpos, n_chunks)

    WD = W_word.shape[1]
    CD = fp_flat.shape[1]
    return word_flat.reshape(B, L, WD), fp_flat.reshape(B, L, CD)


# speedup vs baseline: 5.5872x; 1.0129x over previous
import functools

import jax
import jax.numpy as jnp
from jax import lax
from jax.experimental import pallas as pl
from jax.experimental.pallas import tpu as pltpu
from jax.experimental.pallas import tpu_sc as plsc

NC, NS = 2, 16
NW = NC * NS
SUB = 128
SUBS_PER_CHUNK = 4
CHUNK = SUB * SUBS_PER_CHUNK


@functools.partial(jax.jit, static_argnames=("n_chunks",))
def _sc_embed(w2, f2, p2, r2, W_word, W_field, W_pos, W_rpos, n_chunks):
    N = w2.shape[0] * SUB
    WD = W_word.shape[1]
    FD = W_field.shape[1]
    PD = W_pos.shape[1]
    RD = W_rpos.shape[1]
    CD = FD + PD + RD

    mesh = plsc.VectorSubcoreMesh(
        core_axis_name="c", subcore_axis_name="s", num_cores=NC, num_subcores=NS
    )

    @functools.partial(
        pl.kernel,
        mesh=mesh,
        out_type=[
            jax.ShapeDtypeStruct((N, WD), jnp.float32),
            jax.ShapeDtypeStruct((N, CD), jnp.float32),
        ],
        scratch_types=[
            pltpu.VMEM((2, SUBS_PER_CHUNK, SUB), jnp.int32),
            pltpu.VMEM((2, SUBS_PER_CHUNK, SUB), jnp.int32),
            pltpu.VMEM((2, SUBS_PER_CHUNK, SUB), jnp.int32),
            pltpu.VMEM((2, SUBS_PER_CHUNK, SUB), jnp.int32),
            pltpu.VMEM((2, CHUNK, WD), jnp.float32),
            pltpu.VMEM((2, CHUNK, FD), jnp.float32),
            pltpu.VMEM((2, CHUNK, PD), jnp.float32),
            pltpu.VMEM((2, CHUNK, RD), jnp.float32),
            pltpu.SemaphoreType.DMA,
            pltpu.SemaphoreType.DMA,
            pltpu.SemaphoreType.DMA,
            pltpu.SemaphoreType.DMA,
            pltpu.SemaphoreType.DMA,
        ],
        compiler_params=pltpu.CompilerParams(use_tc_tiling_on_sc=False),
    )
    def k(words_h, fields_h, pos_h, rpos_h, Ww, Wf, Wp, Wr,
          word_out, fp_out, widx, fidx, pidx, ridx,
          wrows, frows, prows, rrows,
          sem_i0, sem_i1, sem_g0, sem_g1, sem_w):
        wid = lax.axis_index("s") * NC + lax.axis_index("c")
        row0 = wid * (n_chunks * SUBS_PER_CHUNK)

        def fire_idx(b, c, sem):
            r = row0 + c * SUBS_PER_CHUNK
            return [
                pltpu.async_copy(words_h.at[pl.ds(r, SUBS_PER_CHUNK)], widx.at[b], sem),
                pltpu.async_copy(fields_h.at[pl.ds(r, SUBS_PER_CHUNK)], fidx.at[b], sem),
                pltpu.async_copy(pos_h.at[pl.ds(r, SUBS_PER_CHUNK)], pidx.at[b], sem),
                pltpu.async_copy(rpos_h.at[pl.ds(r, SUBS_PER_CHUNK)], ridx.at[b], sem),
            ]

        def fire_gathers(b, sem):
            cps = []
            for j in range(SUBS_PER_CHUNK):
                o = j * SUB
                cps.append(pltpu.async_copy(
                    Ww.at[widx.at[b, j]], wrows.at[b, pl.ds(o, SUB)], sem))
                cps.append(pltpu.async_copy(
                    Wf.at[fidx.at[b, j]], frows.at[b, pl.ds(o, SUB)], sem))
                cps.append(pltpu.async_copy(
                    Wp.at[pidx.at[b, j]], prows.at[b, pl.ds(o, SUB)], sem))
                cps.append(pltpu.async_copy(
                    Wr.at[ridx.at[b, j]], rrows.at[b, pl.ds(o, SUB)], sem))
            return cps

        def fire_writes(b, c):
            base = (row0 + c * SUBS_PER_CHUNK) * SUB
            pltpu.async_copy(
                wrows.at[b], word_out.at[pl.ds(base, CHUNK)], sem_w)
            pltpu.async_copy(
                frows.at[b], fp_out.at[pl.ds(base, CHUNK), pl.ds(0, FD)], sem_w)
            pltpu.async_copy(
                prows.at[b], fp_out.at[pl.ds(base, CHUNK), pl.ds(FD, PD)], sem_w)
            pltpu.async_copy(
                rrows.at[b], fp_out.at[pl.ds(base, CHUNK), pl.ds(FD + PD, RD)], sem_w)

        def drain_writes():
            for b in range(2):
                pltpu.make_async_copy(
                    wrows.at[b], word_out.at[pl.ds(0, CHUNK)], sem_w).wait()
                pltpu.make_async_copy(
                    frows.at[b], fp_out.at[pl.ds(0, CHUNK), pl.ds(0, FD)], sem_w).wait()
                pltpu.make_async_copy(
                    prows.at[b], fp_out.at[pl.ds(0, CHUNK), pl.ds(FD, PD)], sem_w).wait()
                pltpu.make_async_copy(
                    rrows.at[b], fp_out.at[pl.ds(0, CHUNK), pl.ds(FD + PD, RD)], sem_w).wait()

        def body(g, carry):
            c0 = 2 * g
            c1 = 2 * g + 1
            i0 = fire_idx(0, c0, sem_i0)
            i1 = fire_idx(1, c1, sem_i1)
            for cp in i0:
                cp.wait()
            @pl.when(g > 0)
            def _():
                drain_writes()
            g0 = fire_gathers(0, sem_g0)
            for cp in i1:
                cp.wait()
            g1 = fire_gathers(1, sem_g1)
            for cp in g0:
                cp.wait()
            fire_writes(0, c0)
            for cp in g1:
                cp.wait()
            fire_writes(1, c1)
            return carry

        lax.fori_loop(0, n_chunks // 2, body, 0)
        drain_writes()

    return k(w2, f2, p2, r2, W_word, W_field, W_pos, W_rpos)


def kernel(words, fields, pos, rpos, W_word, W_field, W_pos, W_rpos):
    B, L = words.shape
    N = B * L
    assert N % (NW * CHUNK * 2) == 0
    n_chunks = N // (NW * CHUNK)

    def prep(ix):
        return ix.reshape(N // SUB, SUB).astype(jnp.int32)

    word_flat, fp_flat = _sc_embed(
        prep(words), prep(fields), prep(pos), prep(rpos),
        W_word, W_field, W_pos, W_rpos, n_chunks)

    WD = W_word.shape[1]
    CD = fp_flat.shape[1]
    return word_flat.reshape(B, L, WD), fp_flat.reshape(B, L, CD)


# speedup vs baseline: 7.0142x; 1.2554x over previous
import functools

import jax
import jax.numpy as jnp
from jax import lax
from jax.experimental import pallas as pl
from jax.experimental.pallas import tpu as pltpu
from jax.experimental.pallas import tpu_sc as plsc

NC, NS = 2, 16
NW = NC * NS
LANES = 16

B, L = 4096, 200
LBLK, BBLK = L // 8, B // 128
PAIRS = LBLK * BBLK
PAIRS_PER_W = PAIRS // NW
WD = 32
FV, FD = 1000, 16
PV, PD = 200, 16
CD = FD + 2 * PD


@jax.jit
def _sc_embed(w4, f4, p4, r4, Wwd, Wf1, Wp1, Wr1):
    mesh = plsc.VectorSubcoreMesh(
        core_axis_name="c", subcore_axis_name="s", num_cores=NC, num_subcores=NS
    )

    @functools.partial(
        pl.kernel,
        mesh=mesh,
        out_type=[
            jax.ShapeDtypeStruct((L * WD * B,), jnp.float32),
            jax.ShapeDtypeStruct((L, CD // 8, BBLK, 8, 128), jnp.float32),
        ],
        scratch_types=[
            pltpu.VMEM((8, 128), jnp.int32),
            pltpu.VMEM((8, 128), jnp.int32),
            pltpu.VMEM((8, 128), jnp.int32),
            pltpu.VMEM((8, 128), jnp.int32),
            pltpu.VMEM((1024, WD), jnp.float32),
            pltpu.VMEM((FV * FD,), jnp.float32),
            pltpu.VMEM((PV * PD,), jnp.float32),
            pltpu.VMEM((PV * PD,), jnp.float32),
            pltpu.VMEM((8 * WD * 128,), jnp.float32),
            pltpu.VMEM((CD // 8, 4, 8, 128), jnp.float32),
            pltpu.SemaphoreType.DMA,
            pltpu.SemaphoreType.DMA,
            pltpu.SemaphoreType.DMA,
        ],
        compiler_params=pltpu.CompilerParams(
            use_tc_tiling_on_sc=False, needs_layout_passes=False),
    )
    def k(w4_h, f4_h, p4_h, r4_h, Ww_h, Wf_h, Wp_h, Wr_h,
          word5, fp5, widx, fidx, pidx, ridx, wrows, tabf, tabp, tabr,
          wT, fpT, sem_i, sem_g, sem_w):
        wid = lax.axis_index("s") * NC + lax.axis_index("c")

        cp_f = pltpu.async_copy(Wf_h, tabf, sem_i)
        cp_p = pltpu.async_copy(Wp_h, tabp, sem_i)
        cp_r = pltpu.async_copy(Wr_h, tabr, sem_i)
        cp_f.wait()
        cp_p.wait()
        cp_r.wait()

        iota = lax.iota(jnp.int32, LANES)

        def pair_body(pp, carry):
            pid = wid * PAIRS_PER_W + pp
            lblk = pid // BBLK
            bblk = pid % BBLK

            i_cps = [
                pltpu.async_copy(w4_h.at[lblk, bblk], widx, sem_i),
                pltpu.async_copy(f4_h.at[lblk, bblk], fidx, sem_i),
                pltpu.async_copy(p4_h.at[lblk, bblk], pidx, sem_i),
                pltpu.async_copy(r4_h.at[lblk, bblk], ridx, sem_i),
            ]
            for cp in i_cps:
                cp.wait()
            g_cps = [
                pltpu.async_copy(
                    Ww_h.at[widx.at[j]], wrows.at[pl.ds(128 * j, 128)], sem_g)
                for j in range(8)
            ]

            fp_w = []
            for h in range(2):
                def fp_lin(i, c2, h=h):
                    lin = 4 * h + i
                    for j in range(8):
                        fv = fidx[lin, pl.ds(16 * j, 16)]
                        pv = pidx[lin, pl.ds(16 * j, 16)]
                        rv = ridx[lin, pl.ds(16 * j, 16)]
                        for dl in range(FD):
                            x = plsc.load_gather(tabf, [fv + dl * FV])
                            fpT[dl // 8, i, dl % 8, pl.ds(16 * j, 16)] = x
                        for dl in range(PD):
                            x = plsc.load_gather(tabp, [pv + dl * PV])
                            d = FD + dl
                            fpT[d // 8, i, d % 8, pl.ds(16 * j, 16)] = x
                        for dl in range(PD):
                            x = plsc.load_gather(tabr, [rv + dl * PV])
                            d = FD + PD + dl
                            fpT[d // 8, i, d % 8, pl.ds(16 * j, 16)] = x
                    return c2

                for cp in fp_w:
                    cp.wait()
                fp_w = []
                lax.fori_loop(0, 4, fp_lin, 0)
                for dblk in range(CD // 8):
                    fp_w.append(pltpu.async_copy(
                        fpT.at[dblk],
                        fp5.at[pl.ds(8 * lblk + 4 * h, 4), dblk, bblk], sem_w))

            for cp in g_cps:
                cp.wait()

            clo = (iota // 8) * 1024 + (iota % 8) * 128
            chi = clo + 2 * 1024

            def w_lin(lin, c2):
                for bin_ in range(128):
                    tok = 128 * lin + bin_
                    xlo = wrows[tok, pl.ds(0, 16)]
                    xhi = wrows[tok, pl.ds(16, 16)]
                    base = lin * (WD * 128) + bin_
                    plsc.store_scatter(wT, [clo + base], xlo)
                    plsc.store_scatter(wT, [chi + base], xhi)
                return c2

            lax.fori_loop(0, 8, w_lin, 0)

            w_w = []
            wbase = (8 * lblk * (WD // 8) * BBLK + bblk) * 1024
            for lin in range(8):
                for dblk in range(WD // 8):
                    src = wT.at[pl.ds(lin * (WD * 128) + dblk * 1024, 1024)]
                    off = wbase + (lin * (WD // 8) * BBLK + dblk * BBLK) * 1024
                    w_w.append(pltpu.async_copy(
                        src, word5.at[pl.ds(off, 1024)], sem_w))
            for cp in fp_w:
                cp.wait()
            for cp in w_w:
                cp.wait()
            return carry

        lax.fori_loop(0, PAIRS_PER_W, pair_body, 0)

    return k(w4, f4, p4, r4, Wwd, Wf1, Wp1, Wr1)


def kernel(words, fields, pos, rpos, W_word, W_field, W_pos, W_rpos):
    def view4(ix):
        return ix.T.reshape(LBLK, 8, BBLK, 128).transpose(0, 2, 1, 3).astype(jnp.int32)

    word5, fp5 = _sc_embed(
        view4(words), view4(fields), view4(pos), view4(rpos),
        W_word,
        W_field.T.reshape(-1),
        W_pos.T.reshape(-1),
        W_rpos.T.reshape(-1),
    )
    word = (word5.reshape(L, WD // 8, BBLK, 8, 128)
            .transpose(2, 4, 0, 1, 3).reshape(B, L, WD))
    fp = fp5.transpose(2, 4, 0, 1, 3).reshape(B, L, CD)
    return word, fp


# speedup vs baseline: 7.2911x; 1.0395x over previous
import functools

import jax
import jax.numpy as jnp
from jax import lax
from jax.experimental import pallas as pl
from jax.experimental.pallas import tpu as pltpu
from jax.experimental.pallas import tpu_sc as plsc

NC, NS = 2, 16
NW = NC * NS
LANES = 16

B, L = 4096, 200
LBLK, BBLK = L // 8, B // 128
PAIRS = LBLK * BBLK
PAIRS_PER_W = PAIRS // NW
WD = 32
FV, FD = 1000, 16
PV, PD = 200, 16
CD = FD + 2 * PD
FS, PS = 1024, 256


@jax.jit
def _sc_embed(w4, f4, p4, r4, Wwd, Wf2, Wp2, Wr2):
    mesh = plsc.VectorSubcoreMesh(
        core_axis_name="c", subcore_axis_name="s", num_cores=NC, num_subcores=NS
    )

    @functools.partial(
        pl.kernel,
        mesh=mesh,
        out_type=[
            jax.ShapeDtypeStruct((L * WD * B,), jnp.float32),
            jax.ShapeDtypeStruct((L, CD // 8, BBLK, 8, 128), jnp.float32),
        ],
        scratch_types=[
            pltpu.VMEM((2, 8, 128), jnp.int32),
            pltpu.VMEM((2, 8, 128), jnp.int32),
            pltpu.VMEM((2, 8, 128), jnp.int32),
            pltpu.VMEM((2, 8, 128), jnp.int32),
            pltpu.VMEM((1024, WD), jnp.float32),
            pltpu.VMEM((FD * FS,), jnp.float32),
            pltpu.VMEM((PD * PS,), jnp.float32),
            pltpu.VMEM((PD * PS,), jnp.float32),
            pltpu.VMEM((8 * WD * 128,), jnp.float32),
            pltpu.VMEM((CD // 8, 4, 8, 128), jnp.float32),
            pltpu.SemaphoreType.DMA,
            pltpu.SemaphoreType.DMA,
            pltpu.SemaphoreType.DMA,
            pltpu.SemaphoreType.DMA,
        ],
        compiler_params=pltpu.CompilerParams(
            use_tc_tiling_on_sc=False, needs_layout_passes=False),
    )
    def k(w4_h, f4_h, p4_h, r4_h, Ww_h, Wf_h, Wp_h, Wr_h,
          word5, fp5, widx, fidx, pidx, ridx, wrows, tabf, tabp, tabr,
          wT, fpT, sem_i, sem_g, sem_wf, sem_ww):
        wid = lax.axis_index("s") * NC + lax.axis_index("c")

        t_cps = []
        for dl in range(FD):
            t_cps.append(pltpu.async_copy(
                Wf_h.at[dl], tabf.at[pl.ds(dl * FS, FV)], sem_i))
        for dl in range(PD):
            t_cps.append(pltpu.async_copy(
                Wp_h.at[dl], tabp.at[pl.ds(dl * PS, PV)], sem_i))
            t_cps.append(pltpu.async_copy(
                Wr_h.at[dl], tabr.at[pl.ds(dl * PS, PV)], sem_i))
        for cp in t_cps:
            cp.wait()

        iota = lax.iota(jnp.int32, LANES)
        clo = (iota // 8) * 1024 + (iota % 8) * 128
        chi = clo + 2 * 1024

        def fire_idx(pid, sel):
            lblk = pid // BBLK
            bblk = pid % BBLK
            pltpu.async_copy(w4_h.at[lblk, bblk], widx.at[sel], sem_i)
            pltpu.async_copy(f4_h.at[lblk, bblk], fidx.at[sel], sem_i)
            pltpu.async_copy(p4_h.at[lblk, bblk], pidx.at[sel], sem_i)
            pltpu.async_copy(r4_h.at[lblk, bblk], ridx.at[sel], sem_i)

        def drain_idx():
            for r in (widx, fidx, pidx, ridx):
                pltpu.make_async_copy(
                    w4_h.at[0, 0], r.at[0], sem_i).wait()

        def drain_fp_half():
            for dblk in range(CD // 8):
                pltpu.make_async_copy(
                    fpT.at[dblk], fp5.at[pl.ds(0, 4), 0, 0], sem_wf).wait()

        def drain_word():
            pltpu.make_async_copy(
                wT, word5.at[pl.ds(0, 8 * WD * 128)], sem_ww).wait()

        fire_idx(wid * PAIRS_PER_W, 0)

        def pair_body(pp, carry):
            sel = pp % 2
            pid = wid * PAIRS_PER_W + pp
            lblk = pid // BBLK
            bblk = pid % BBLK

            drain_idx()
            g_cps = [
                pltpu.async_copy(
                    Ww_h.at[widx.at[sel, j]],
                    wrows.at[pl.ds(128 * j, 128)], sem_g)
                for j in range(8)
            ]
            nxt = jnp.where(pp + 1 < PAIRS_PER_W, pid + 1, pid)
            fire_idx(nxt, 1 - sel)

            for h in range(2):
                def fp_lin(i, c2, h=h):
                    lin = 4 * h + i
                    for j in range(8):
                        fv = fidx[sel, lin, pl.ds(16 * j, 16)]
                        pv = pidx[sel, lin, pl.ds(16 * j, 16)]
                        rv = ridx[sel, lin, pl.ds(16 * j, 16)]
                        for dl in range(FD):
                            x = plsc.load_gather(
                                tabf.at[pl.ds(dl * FS, FS)], [fv])
                            fpT[dl // 8, i, dl % 8, pl.ds(16 * j, 16)] = x
                        for dl in range(PD):
                            x = plsc.load_gather(
                                tabp.at[pl.ds(dl * PS, PS)], [pv])
                            d = FD + dl
                            fpT[d // 8, i, d % 8, pl.ds(16 * j, 16)] = x
                        for dl in range(PD):
                            x = plsc.load_gather(
                                tabr.at[pl.ds(dl * PS, PS)], [rv])
                            d = FD + PD + dl
                            fpT[d // 8, i, d % 8, pl.ds(16 * j, 16)] = x
                    return c2

                @pl.when(jnp.logical_or(pp > 0, h > 0))
                def _():
                    drain_fp_half()
                lax.fori_loop(0, 4, fp_lin, 0)
                for dblk in range(CD // 8):
                    pltpu.async_copy(
                        fpT.at[dblk],
                        fp5.at[pl.ds(8 * lblk + 4 * h, 4), dblk, bblk], sem_wf)

            for cp in g_cps:
                cp.wait()

            @pl.when(pp > 0)
            def _():
                drain_word()

            def w_lin(lin, c2):
                for bin_ in range(128):
                    tok = 128 * lin + bin_
                    xlo = wrows[tok, pl.ds(0, 16)]
                    xhi = wrows[tok, pl.ds(16, 16)]
                    base = lin * (WD * 128) + bin_
                    plsc.store_scatter(wT, [clo + base], xlo)
                    plsc.store_scatter(wT, [chi + base], xhi)
                return c2

            lax.fori_loop(0, 8, w_lin, 0)

            wbase = (8 * lblk * (WD // 8) * BBLK + bblk) * 1024
            for lin in range(8):
                for dblk in range(WD // 8):
                    src = wT.at[pl.ds(lin * (WD * 128) + dblk * 1024, 1024)]
                    off = wbase + (lin * (WD // 8) * BBLK + dblk * BBLK) * 1024
                    pltpu.async_copy(src, word5.at[pl.ds(off, 1024)], sem_ww)
            return carry

        lax.fori_loop(0, PAIRS_PER_W, pair_body, 0)
        drain_idx()
        drain_fp_half()
        drain_word()

    return k(w4, f4, p4, r4, Wwd, Wf2, Wp2, Wr2)


def kernel(words, fields, pos, rpos, W_word, W_field, W_pos, W_rpos):
    def view4(ix):
        return ix.T.reshape(LBLK, 8, BBLK, 128).transpose(0, 2, 1, 3).astype(jnp.int32)

    word5, fp5 = _sc_embed(
        view4(words), view4(fields), view4(pos), view4(rpos),
        W_word, W_field.T, W_pos.T, W_rpos.T)
    word = (word5.reshape(L, WD // 8, BBLK, 8, 128)
            .transpose(2, 4, 0, 1, 3).reshape(B, L, WD))
    fp = fp5.transpose(2, 4, 0, 1, 3).reshape(B, L, CD)
    return word, fp


# speedup vs baseline: 8.1450x; 1.1171x over previous
import functools

import jax
import jax.numpy as jnp
from jax import lax
from jax.experimental import pallas as pl
from jax.experimental.pallas import tpu as pltpu
from jax.experimental.pallas import tpu_sc as plsc

NC, NS = 2, 16
NW = NC * NS
LANES = 16

B, L = 4096, 200
LBLK, BBLK = L // 8, B // 128
PAIRS = LBLK * BBLK
PAIRS_PER_W = PAIRS // NW
WD = 32
FV, FD = 1000, 16
PV, PD = 200, 16
CD = FD + 2 * PD
FS, PS = 1024, 256


@jax.jit
def _sc_embed(w4, f4, p4, r4, Wwd, Wf2, Wp2, Wr2):
    mesh = plsc.VectorSubcoreMesh(
        core_axis_name="c", subcore_axis_name="s", num_cores=NC, num_subcores=NS
    )

    @functools.partial(
        pl.kernel,
        mesh=mesh,
        out_type=[
            jax.ShapeDtypeStruct((L * WD * B,), jnp.float32),
            jax.ShapeDtypeStruct((L, CD // 8, BBLK, 8, 128), jnp.float32),
        ],
        scratch_types=[
            pltpu.VMEM((2, 8, 128), jnp.int32),
            pltpu.VMEM((2, 8, 128), jnp.int32),
            pltpu.VMEM((2, 8, 128), jnp.int32),
            pltpu.VMEM((2, 8, 128), jnp.int32),
            pltpu.VMEM((1024, WD), jnp.float32),
            pltpu.VMEM((FD * FS,), jnp.float32),
            pltpu.VMEM((PD * PS,), jnp.float32),
            pltpu.VMEM((PD * PS,), jnp.float32),
            pltpu.VMEM((8 * WD * 128,), jnp.float32),
            pltpu.VMEM((CD // 8, 4, 8, 128), jnp.float32),
            pltpu.SemaphoreType.DMA,
            pltpu.SemaphoreType.DMA,
            pltpu.SemaphoreType.DMA,
            pltpu.SemaphoreType.DMA,
        ],
        compiler_params=pltpu.CompilerParams(
            use_tc_tiling_on_sc=False, needs_layout_passes=False),
    )
    def k(w4_h, f4_h, p4_h, r4_h, Ww_h, Wf_h, Wp_h, Wr_h,
          word5, fp5, widx, fidx, pidx, ridx, wrows, tabf, tabp, tabr,
          wT, fpT, sem_i, sem_g, sem_wf, sem_ww):
        wid = lax.axis_index("s") * NC + lax.axis_index("c")

        t_cps = []
        for dl in range(FD):
            t_cps.append(pltpu.async_copy(
                Wf_h.at[dl], tabf.at[pl.ds(dl * FS, FV)], sem_i))
        for dl in range(PD):
            t_cps.append(pltpu.async_copy(
                Wp_h.at[dl], tabp.at[pl.ds(dl * PS, PV)], sem_i))
            t_cps.append(pltpu.async_copy(
                Wr_h.at[dl], tabr.at[pl.ds(dl * PS, PV)], sem_i))
        for cp in t_cps:
            cp.wait()

        iota = lax.iota(jnp.int32, LANES)
        clo = (iota // 8) * 1024 + (iota % 8) * 128
        chi = clo + 2 * 1024

        def fire_idx(pid, sel):
            lblk = pid // BBLK
            bblk = pid % BBLK
            pltpu.async_copy(w4_h.at[lblk, bblk], widx.at[sel], sem_i)
            pltpu.async_copy(f4_h.at[lblk, bblk], fidx.at[sel], sem_i)
            pltpu.async_copy(p4_h.at[lblk, bblk], pidx.at[sel], sem_i)
            pltpu.async_copy(r4_h.at[lblk, bblk], ridx.at[sel], sem_i)

        def drain_idx():
            for r in (widx, fidx, pidx, ridx):
                pltpu.make_async_copy(
                    w4_h.at[0, 0], r.at[0], sem_i).wait()

        def drain_fp_half():
            for dblk in range(CD // 8):
                pltpu.make_async_copy(
                    fpT.at[dblk], fp5.at[pl.ds(0, 4), 0, 0], sem_wf).wait()

        def drain_word():
            pltpu.make_async_copy(
                wT, word5.at[pl.ds(0, 8 * WD * 128)], sem_ww).wait()

        fire_idx(wid * PAIRS_PER_W, 0)

        def pair_body(pp, carry):
            sel = pp % 2
            pid = wid * PAIRS_PER_W + pp
            lblk = pid // BBLK
            bblk = pid % BBLK

            drain_idx()
            g_cps = [
                pltpu.async_copy(
                    Ww_h.at[widx.at[sel, j]],
                    wrows.at[pl.ds(128 * j, 128)], sem_g)
                for j in range(8)
            ]
            nxt = jnp.where(pp + 1 < PAIRS_PER_W, pid + 1, pid)
            fire_idx(nxt, 1 - sel)

            for h in range(2):
                def fp_lin(i, h=h):
                    lin = 4 * h + i
                    for j in range(8):
                        fv = fidx[sel, lin, pl.ds(16 * j, 16)]
                        pv = pidx[sel, lin, pl.ds(16 * j, 16)]
                        rv = ridx[sel, lin, pl.ds(16 * j, 16)]
                        for dl in range(FD):
                            x = plsc.load_gather(
                                tabf.at[pl.ds(dl * FS, FS)], [fv])
                            fpT[dl // 8, i, dl % 8, pl.ds(16 * j, 16)] = x
                        for dl in range(PD):
                            x = plsc.load_gather(
                                tabp.at[pl.ds(dl * PS, PS)], [pv])
                            d = FD + dl
                            fpT[d // 8, i, d % 8, pl.ds(16 * j, 16)] = x
                        for dl in range(PD):
                            x = plsc.load_gather(
                                tabr.at[pl.ds(dl * PS, PS)], [rv])
                            d = FD + PD + dl
                            fpT[d // 8, i, d % 8, pl.ds(16 * j, 16)] = x

                @pl.when(jnp.logical_or(pp > 0, h > 0))
                def _():
                    drain_fp_half()
                plsc.parallel_loop(0, 4)(fp_lin)
                for dblk in range(CD // 8):
                    pltpu.async_copy(
                        fpT.at[dblk],
                        fp5.at[pl.ds(8 * lblk + 4 * h, 4), dblk, bblk], sem_wf)

            for cp in g_cps:
                cp.wait()

            @pl.when(pp > 0)
            def _():
                drain_word()

            def w_lin(lin):
                for bin_ in range(128):
                    tok = 128 * lin + bin_
                    xlo = wrows[tok, pl.ds(0, 16)]
                    xhi = wrows[tok, pl.ds(16, 16)]
                    base = lin * (WD * 128) + bin_
                    plsc.store_scatter(wT, [clo + base], xlo)
                    plsc.store_scatter(wT, [chi + base], xhi)

            plsc.parallel_loop(0, 8)(w_lin)

            wbase = (8 * lblk * (WD // 8) * BBLK + bblk) * 1024
            for lin in range(8):
                for dblk in range(WD // 8):
                    src = wT.at[pl.ds(lin * (WD * 128) + dblk * 1024, 1024)]
                    off = wbase + (lin * (WD // 8) * BBLK + dblk * BBLK) * 1024
                    pltpu.async_copy(src, word5.at[pl.ds(off, 1024)], sem_ww)
            return carry

        lax.fori_loop(0, PAIRS_PER_W, pair_body, 0)
        drain_idx()
        drain_fp_half()
        drain_word()

    return k(w4, f4, p4, r4, Wwd, Wf2, Wp2, Wr2)


def kernel(words, fields, pos, rpos, W_word, W_field, W_pos, W_rpos):
    def view4(ix):
        return ix.T.reshape(LBLK, 8, BBLK, 128).transpose(0, 2, 1, 3).astype(jnp.int32)

    word5, fp5 = _sc_embed(
        view4(words), view4(fields), view4(pos), view4(rpos),
        W_word, W_field.T, W_pos.T, W_rpos.T)
    word = (word5.reshape(L, WD // 8, BBLK, 8, 128)
            .transpose(2, 4, 0, 1, 3).reshape(B, L, WD))
    fp = fp5.transpose(2, 4, 0, 1, 3).reshape(B, L, CD)
    return word, fp


# speedup vs baseline: 9.4670x; 1.1623x over previous
import functools

import jax
import jax.numpy as jnp
from jax import lax
from jax.experimental import pallas as pl
from jax.experimental.pallas import tpu as pltpu
from jax.experimental.pallas import tpu_sc as plsc

NC, NS = 2, 16
NW = NC * NS
LANES = 16

B, L = 4096, 200
LBLK, BBLK = L // 8, B // 128
PAIRS = LBLK * BBLK
PAIRS_PER_W = PAIRS // NW
WD = 32
FV, FD = 1000, 16
PV, PD = 200, 16
CD = FD + 2 * PD
FS, PS = 1024, 256

_MESH = dict(core_axis_name="c", subcore_axis_name="s",
             num_cores=NC, num_subcores=NS)
_PARAMS = pltpu.CompilerParams(
    use_tc_tiling_on_sc=False, needs_layout_passes=False)


@jax.jit
def _sc_fp(f4, p4, r4, Wf2, Wp2, Wr2):
    @functools.partial(
        pl.kernel,
        mesh=plsc.VectorSubcoreMesh(**_MESH),
        out_type=[jax.ShapeDtypeStruct((L, CD // 8, BBLK, 8, 128), jnp.float32)],
        scratch_types=[
            pltpu.VMEM((2, 8, 128), jnp.int32),
            pltpu.VMEM((2, 8, 128), jnp.int32),
            pltpu.VMEM((2, 8, 128), jnp.int32),
            pltpu.VMEM((FD * FS,), jnp.float32),
            pltpu.VMEM((PD * PS,), jnp.float32),
            pltpu.VMEM((PD * PS,), jnp.float32),
            pltpu.VMEM((CD // 8, 8, 8, 128), jnp.float32),
            pltpu.SemaphoreType.DMA,
            pltpu.SemaphoreType.DMA,
        ],
        compiler_params=_PARAMS,
    )
    def k_fp(f4_h, p4_h, r4_h, Wf_h, Wp_h, Wr_h,
             fp5, fidx, pidx, ridx, tabf, tabp, tabr, fpT, sem_i, sem_w):
        wid = lax.axis_index("s") * NC + lax.axis_index("c")

        t_cps = []
        for dl in range(FD):
            t_cps.append(pltpu.async_copy(
                Wf_h.at[dl], tabf.at[pl.ds(dl * FS, FV)], sem_i))
        for dl in range(PD):
            t_cps.append(pltpu.async_copy(
                Wp_h.at[dl], tabp.at[pl.ds(dl * PS, PV)], sem_i))
            t_cps.append(pltpu.async_copy(
                Wr_h.at[dl], tabr.at[pl.ds(dl * PS, PV)], sem_i))
        for cp in t_cps:
            cp.wait()

        def fire_idx(pid, sel):
            lblk = pid // BBLK
            bblk = pid % BBLK
            pltpu.async_copy(f4_h.at[lblk, bblk], fidx.at[sel], sem_i)
            pltpu.async_copy(p4_h.at[lblk, bblk], pidx.at[sel], sem_i)
            pltpu.async_copy(r4_h.at[lblk, bblk], ridx.at[sel], sem_i)

        def drain_idx():
            for r in (fidx, pidx, ridx):
                pltpu.make_async_copy(f4_h.at[0, 0], r.at[0], sem_i).wait()

        def drain_writes():
            for dblk in range(CD // 8):
                pltpu.make_async_copy(
                    fpT.at[dblk], fp5.at[pl.ds(0, 8), 0, 0], sem_w).wait()

        fire_idx(wid * PAIRS_PER_W, 0)

        def pair_body(pp, carry):
            sel = pp % 2
            pid = wid * PAIRS_PER_W + pp
            lblk = pid // BBLK
            bblk = pid % BBLK

            drain_idx()
            nxt = jnp.where(pp + 1 < PAIRS_PER_W, pid + 1, pid)
            fire_idx(nxt, 1 - sel)

            def fp_lin(lin):
                for j in range(8):
                    fv = fidx[sel, lin, pl.ds(16 * j, 16)]
                    pv = pidx[sel, lin, pl.ds(16 * j, 16)]
                    rv = ridx[sel, lin, pl.ds(16 * j, 16)]
                    for dl in range(FD):
                        x = plsc.load_gather(tabf.at[pl.ds(dl * FS, FS)], [fv])
                        fpT[dl // 8, lin, dl % 8, pl.ds(16 * j, 16)] = x
                    for dl in range(PD):
                        x = plsc.load_gather(tabp.at[pl.ds(dl * PS, PS)], [pv])
                        d = FD + dl
                        fpT[d // 8, lin, d % 8, pl.ds(16 * j, 16)] = x
                    for dl in range(PD):
                        x = plsc.load_gather(tabr.at[pl.ds(dl * PS, PS)], [rv])
                        d = FD + PD + dl
                        fpT[d // 8, lin, d % 8, pl.ds(16 * j, 16)] = x

            @pl.when(pp > 0)
            def _():
                drain_writes()
            plsc.parallel_loop(0, 8)(fp_lin)
            for dblk in range(CD // 8):
                pltpu.async_copy(
                    fpT.at[dblk],
                    fp5.at[pl.ds(8 * lblk, 8), dblk, bblk], sem_w)
            return carry

        lax.fori_loop(0, PAIRS_PER_W, pair_body, 0)
        drain_idx()
        drain_writes()

    return k_fp(f4, p4, r4, Wf2, Wp2, Wr2)


@jax.jit
def _sc_word(w4, Wwd):
    @functools.partial(
        pl.kernel,
        mesh=plsc.VectorSubcoreMesh(**_MESH),
        out_type=[jax.ShapeDtypeStruct((L * WD * B,), jnp.float32)],
        scratch_types=[
            pltpu.VMEM((2, 8, 128), jnp.int32),
            pltpu.VMEM((1024, WD), jnp.float32),
            pltpu.VMEM((8 * WD * 128,), jnp.float32),
            pltpu.SemaphoreType.DMA,
            pltpu.SemaphoreType.DMA,
            pltpu.SemaphoreType.DMA,
        ],
        compiler_params=_PARAMS,
    )
    def k_word(w4_h, Ww_h, word5, widx, wrows, wT, sem_i, sem_g, sem_w):
        wid = lax.axis_index("s") * NC + lax.axis_index("c")
        iota = lax.iota(jnp.int32, LANES)
        clo = (iota // 8) * 1024 + (iota % 8) * 128
        chi = clo + 2 * 1024

        def fire_idx(pid, sel):
            lblk = pid // BBLK
            bblk = pid % BBLK
            pltpu.async_copy(w4_h.at[lblk, bblk], widx.at[sel], sem_i)

        def drain_idx():
            pltpu.make_async_copy(w4_h.at[0, 0], widx.at[0], sem_i).wait()

        def drain_writes():
            pltpu.make_async_copy(
                wT, word5.at[pl.ds(0, 8 * WD * 128)], sem_w).wait()

        fire_idx(wid * PAIRS_PER_W, 0)

        def pair_body(pp, carry):
            sel = pp % 2
            pid = wid * PAIRS_PER_W + pp
            lblk = pid // BBLK
            bblk = pid % BBLK

            drain_idx()
            g_cps = [
                pltpu.async_copy(
                    Ww_h.at[widx.at[sel, j]],
                    wrows.at[pl.ds(128 * j, 128)], sem_g)
                for j in range(8)
            ]
            nxt = jnp.where(pp + 1 < PAIRS_PER_W, pid + 1, pid)
            fire_idx(nxt, 1 - sel)

            for cp in g_cps:
                cp.wait()

            @pl.when(pp > 0)
            def _():
                drain_writes()

            def w_lin(lin):
                for bin_ in range(128):
                    tok = 128 * lin + bin_
                    xlo = wrows[tok, pl.ds(0, 16)]
                    xhi = wrows[tok, pl.ds(16, 16)]
                    base = lin * (WD * 128) + bin_
                    plsc.store_scatter(wT, [clo + base], xlo)
                    plsc.store_scatter(wT, [chi + base], xhi)

            plsc.parallel_loop(0, 8)(w_lin)

            wbase = (8 * lblk * (WD // 8) * BBLK + bblk) * 1024
            for lin in range(8):
                for dblk in range(WD // 8):
                    src = wT.at[pl.ds(lin * (WD * 128) + dblk * 1024, 1024)]
                    off = wbase + (lin * (WD // 8) * BBLK + dblk * BBLK) * 1024
                    pltpu.async_copy(src, word5.at[pl.ds(off, 1024)], sem_w)
            return carry

        lax.fori_loop(0, PAIRS_PER_W, pair_body, 0)
        drain_idx()
        drain_writes()

    return k_word(w4, Wwd)


def kernel(words, fields, pos, rpos, W_word, W_field, W_pos, W_rpos):
    def view4(ix):
        return ix.T.reshape(LBLK, 8, BBLK, 128).transpose(0, 2, 1, 3).astype(jnp.int32)

    (fp5,) = _sc_fp(view4(fields), view4(pos), view4(rpos),
                    W_field.T, W_pos.T, W_rpos.T)
    (word5,) = _sc_word(view4(words), W_word)
    word = (word5.reshape(L, WD // 8, BBLK, 8, 128)
            .transpose(2, 4, 0, 1, 3).reshape(B, L, WD))
    fp = fp5.transpose(2, 4, 0, 1, 3).reshape(B, L, CD)
    return word, fp
